# baseline (speedup 1.0000x reference)
import functools

import jax
import jax.numpy as jnp
from jax.experimental import pallas as pl


def _chamfer_body(x1_ref, x2t_ref, d1_ref, d2_ref, *, bn: int):
    i = pl.program_id(1)
    x1b = x1_ref[0]
    x2b = x2t_ref[0]

    sq1 = jnp.sum(x1b * x1b, axis=1, keepdims=True)
    sq2 = jnp.sum(x2b * x2b, axis=0, keepdims=True)
    cross = jax.lax.dot_general(
        x1b.astype(jnp.bfloat16),
        x2b.astype(jnp.bfloat16),
        (((1,), (0,)), ((), ())),
        preferred_element_type=jnp.float32,
    )
    d = (sq1 + sq2) - 2.0 * cross

    d1_ref[0, 0, pl.ds(i * bn, bn)] = jnp.min(d, axis=1)

    colmin = jnp.min(d, axis=0)

    @pl.when(i == 0)
    def _init():
        d2_ref[0, 0, :] = colmin

    @pl.when(i > 0)
    def _acc():
        d2_ref[0, 0, :] = jnp.minimum(d2_ref[0, 0, :], colmin)


@jax.jit
def kernel(input1, input2):
    B, N, _ = input1.shape
    _, M, _ = input2.shape
    bn = 256
    x2t = input2.transpose(0, 2, 1)

    d1, d2 = pl.pallas_call(
        functools.partial(_chamfer_body, bn=bn),
        grid=(B, N // bn),
        in_specs=[
            pl.BlockSpec((1, bn, 3), lambda b, i: (b, i, 0)),
            pl.BlockSpec((1, 3, M), lambda b, i: (b, 0, 0)),
        ],
        out_specs=[
            pl.BlockSpec((1, 1, N), lambda b, i: (b, 0, 0)),
            pl.BlockSpec((1, 1, M), lambda b, i: (b, 0, 0)),
        ],
        out_shape=[
            jax.ShapeDtypeStruct((B, 1, N), jnp.float32),
            jax.ShapeDtypeStruct((B, 1, M), jnp.float32),
        ],
    )(input1, x2t)
    return d1.reshape(B, N), d2.reshape(B, M)


# speedup vs baseline: 1.1548x; 1.1548x over previous
import functools

import jax
import jax.numpy as jnp
from jax.experimental import pallas as pl


def _chamfer_body(x1_ref, x2t_ref, d1_ref, d2_ref, *, bn: int):
    i = pl.program_id(1)
    x1b = x1_ref[0]
    x2b = x2t_ref[0]

    sq1 = jnp.sum(x1b * x1b, axis=1, keepdims=True)
    sq2 = jnp.sum(x2b * x2b, axis=0, keepdims=True)
    neg2cross = jax.lax.dot_general(
        (-2.0 * x1b).astype(jnp.bfloat16),
        x2b.astype(jnp.bfloat16),
        (((1,), (0,)), ((), ())),
        preferred_element_type=jnp.float32,
    )
    d = (sq1 + sq2) + neg2cross

    d1_ref[0, 0, pl.ds(i * bn, bn)] = jnp.min(d, axis=1)

    colmin = jnp.min(d, axis=0)

    @pl.when(i == 0)
    def _init():
        d2_ref[0, 0, :] = colmin

    @pl.when(i > 0)
    def _acc():
        d2_ref[0, 0, :] = jnp.minimum(d2_ref[0, 0, :], colmin)


@jax.jit
def kernel(input1, input2):
    B, N, _ = input1.shape
    _, M, _ = input2.shape
    bn = 512
    x2t = input2.transpose(0, 2, 1)

    d1, d2 = pl.pallas_call(
        functools.partial(_chamfer_body, bn=bn),
        grid=(B, N // bn),
        in_specs=[
            pl.BlockSpec((1, bn, 3), lambda b, i: (b, i, 0)),
            pl.BlockSpec((1, 3, M), lambda b, i: (b, 0, 0)),
        ],
        out_specs=[
            pl.BlockSpec((1, 1, N), lambda b, i: (b, 0, 0)),
            pl.BlockSpec((1, 1, M), lambda b, i: (b, 0, 0)),
        ],
        out_shape=[
            jax.ShapeDtypeStruct((B, 1, N), jnp.float32),
            jax.ShapeDtypeStruct((B, 1, M), jnp.float32),
        ],
    )(input1, x2t)
    return d1.reshape(B, N), d2.reshape(B, M)
